# baseline (speedup 1.0000x reference)
import jax
import jax.numpy as jnp
from jax import lax
from jax.experimental import pallas as pl
from jax.experimental.pallas import tpu as pltpu
from jax.experimental.pallas import tpu_sc as plsc

_B = 4096
_L = 20
_D = 64
_LANES = 16
_NC = 2
_NS = 16
_NW = _NC * _NS
_BPW = _B // _NW
_G = _BPW // _LANES
_L2 = 1e-05


def _sc_body(users_hbm, bundles_hbm, uf_hbm, bf_hbm,
             pred_hbm, part_hbm,
             uidx_v, bidx_v, idxt_v, urows_v, brows_v, pbuf_v, pvec_v, sem):
    cid = lax.axis_index("c")
    sid = lax.axis_index("s")
    wid = sid * _NC + cid
    base = wid * _BPW
    iota = lax.iota(jnp.int32, _LANES)

    pltpu.sync_copy(users_hbm.at[pl.ds(base, _BPW)], uidx_v)
    pltpu.sync_copy(bundles_hbm.at[pl.ds(base * _L, _BPW * _L)], bidx_v)

    udma = pltpu.async_copy(uf_hbm.at[uidx_v], urows_v, sem)

    def tbody(l, carry):
        ls = jnp.full((_LANES,), l, jnp.int32)
        for g in range(_G):
            src = (iota + (g * _LANES)) * _L + ls
            idxt_v[pl.ds(l * _BPW + g * _LANES, _LANES)] = (
                plsc.load_gather(bidx_v, [src]))
        return carry
    lax.fori_loop(0, _L, tbody, jnp.int32(0))

    udma.wait()

    def ubody(r, usq):
        for c in range(0, _D, _LANES):
            v = urows_v[r, pl.ds(c, _LANES)]
            usq = usq + v * v
        return usq
    usq = lax.fori_loop(0, _BPW, ubody, jnp.zeros((_LANES,), jnp.float32))

    def body(l, bsq):
        pltpu.async_copy(
            bf_hbm.at[idxt_v.at[pl.ds(l * _BPW, _BPW)]],
            brows_v, sem).wait()
        ls = jnp.full((_LANES,), l, jnp.int32)
        for g in range(_G):
            rows = iota + (g * _LANES)
            acc = jnp.zeros((_LANES,), jnp.float32)
            for j in range(_D):
                js = jnp.full((_LANES,), j, jnp.int32)
                uv = plsc.load_gather(urows_v, [rows, js])
                bv = plsc.load_gather(brows_v, [rows, js])
                acc = acc + uv * bv
                bsq = bsq + bv * bv
            plsc.store_scatter(pbuf_v, [rows * _L + ls], acc)
        return bsq
    bsq = lax.fori_loop(0, _L, body, jnp.zeros((_LANES,), jnp.float32))

    pltpu.sync_copy(pbuf_v, pred_hbm.at[pl.ds(base * _L, _BPW * _L)])

    pvec_v[...] = jnp.float32(_L) * usq + bsq
    pltpu.sync_copy(pvec_v, part_hbm.at[pl.ds(wid * _LANES, _LANES)])


_sc_kernel = pl.kernel(
    _sc_body,
    out_type=[
        jax.ShapeDtypeStruct((_B * _L,), jnp.float32),
        jax.ShapeDtypeStruct((_NW * _LANES,), jnp.float32),
    ],
    mesh=plsc.VectorSubcoreMesh(core_axis_name="c", subcore_axis_name="s"),
    compiler_params=pltpu.CompilerParams(
        needs_layout_passes=False, use_tc_tiling_on_sc=False),
    scratch_types=[
        pltpu.VMEM((_BPW,), jnp.int32),
        pltpu.VMEM((_BPW * _L,), jnp.int32),
        pltpu.VMEM((_L * _BPW,), jnp.int32),
        pltpu.VMEM((_BPW, _D), jnp.float32),
        pltpu.VMEM((_BPW, _D), jnp.float32),
        pltpu.VMEM((_BPW * _L,), jnp.float32),
        pltpu.VMEM((_LANES,), jnp.float32),
        pltpu.SemaphoreType.DMA,
    ],
)


@jax.jit
def kernel(users, bundles, users_feature, bundles_feature):
    pred_flat, parts = _sc_kernel(
        users.reshape(_B), bundles.reshape(_B * _L),
        users_feature, bundles_feature)
    pred = pred_flat.reshape(_B, _L)
    loss = jnp.float32(_L2) * jnp.sum(parts)
    return (pred, loss)


# speedup vs baseline: 1.6823x; 1.6823x over previous
import jax
import jax.numpy as jnp
from jax import lax
from jax.experimental import pallas as pl
from jax.experimental.pallas import tpu as pltpu
from jax.experimental.pallas import tpu_sc as plsc

_B = 4096
_L = 20
_D = 64
_LANES = 16
_NC = 2
_NS = 16
_NW = _NC * _NS
_BPW = _B // _NW
_G = _BPW // _LANES
_L2 = 1e-05


def _sc_body(users_hbm, bundles_hbm, uf_hbm, bf_hbm,
             pred_hbm, part_hbm,
             uidx_v, bidx_v, idxt_v, urows_v, brows_v, pbuf_v, pvec_v,
             usem, bsem):
    cid = lax.axis_index("c")
    sid = lax.axis_index("s")
    wid = sid * _NC + cid
    base = wid * _BPW
    iota = lax.iota(jnp.int32, _LANES)

    pltpu.sync_copy(users_hbm.at[pl.ds(base, _BPW)], uidx_v)
    pltpu.sync_copy(bundles_hbm.at[pl.ds(base * _L, _BPW * _L)], bidx_v)

    udma = pltpu.async_copy(uf_hbm.at[uidx_v], urows_v, usem)

    def tbody(l, carry):
        ls = jnp.full((_LANES,), l, jnp.int32)
        for g in range(_G):
            src = (iota + (g * _LANES)) * _L + ls
            idxt_v[pl.ds(l * _BPW + g * _LANES, _LANES)] = (
                plsc.load_gather(bidx_v, [src]))
        return carry
    lax.fori_loop(0, _L, tbody, jnp.int32(0))

    pltpu.async_copy(
        bf_hbm.at[idxt_v.at[pl.ds(0, _BPW)]],
        brows_v.at[pl.ds(0, _BPW)], bsem.at[0])

    udma.wait()

    def ubody(r, usq):
        for c in range(0, _D, _LANES):
            v = urows_v[r, pl.ds(c, _LANES)]
            usq = usq + v * v
        return usq
    usq = lax.fori_loop(0, _BPW, ubody, jnp.zeros((_LANES,), jnp.float32))

    def body(l, bsq):
        cur = lax.rem(l, 2)
        nxt = 1 - cur

        @pl.when(l + 1 < _L)
        def _prefetch():
            pltpu.async_copy(
                bf_hbm.at[idxt_v.at[pl.ds((l + 1) * _BPW, _BPW)]],
                brows_v.at[pl.ds(nxt * _BPW, _BPW)], bsem.at[nxt])

        pltpu.make_async_copy(
            bf_hbm.at[idxt_v.at[pl.ds(l * _BPW, _BPW)]],
            brows_v.at[pl.ds(cur * _BPW, _BPW)], bsem.at[cur]).wait()

        ls = jnp.full((_LANES,), l, jnp.int32)
        roff = cur * _BPW
        for g in range(_G):
            rows = iota + (g * _LANES)
            brows = rows + roff
            acc = jnp.zeros((_LANES,), jnp.float32)
            for j in range(_D):
                cols = iota + j
                if j + _LANES > _D:
                    cols = lax.bitwise_and(cols, _D - 1)
                uv = plsc.load_gather(urows_v, [rows, cols])
                bv = plsc.load_gather(brows_v, [brows, cols])
                acc = acc + uv * bv
                bsq = bsq + bv * bv
            plsc.store_scatter(pbuf_v, [rows * _L + ls], acc)
        return bsq
    bsq = lax.fori_loop(0, _L, body, jnp.zeros((_LANES,), jnp.float32))

    pltpu.sync_copy(pbuf_v, pred_hbm.at[pl.ds(base * _L, _BPW * _L)])

    pvec_v[...] = jnp.float32(_L) * usq + bsq
    pltpu.sync_copy(pvec_v, part_hbm.at[pl.ds(wid * _LANES, _LANES)])


_sc_kernel = pl.kernel(
    _sc_body,
    out_type=[
        jax.ShapeDtypeStruct((_B * _L,), jnp.float32),
        jax.ShapeDtypeStruct((_NW * _LANES,), jnp.float32),
    ],
    mesh=plsc.VectorSubcoreMesh(core_axis_name="c", subcore_axis_name="s"),
    compiler_params=pltpu.CompilerParams(
        needs_layout_passes=False, use_tc_tiling_on_sc=False),
    scratch_types=[
        pltpu.VMEM((_BPW,), jnp.int32),
        pltpu.VMEM((_BPW * _L,), jnp.int32),
        pltpu.VMEM((_L * _BPW,), jnp.int32),
        pltpu.VMEM((_BPW, _D), jnp.float32),
        pltpu.VMEM((2 * _BPW, _D), jnp.float32),
        pltpu.VMEM((_BPW * _L,), jnp.float32),
        pltpu.VMEM((_LANES,), jnp.float32),
        pltpu.SemaphoreType.DMA,
        pltpu.SemaphoreType.DMA((2,)),
    ],
)


@jax.jit
def kernel(users, bundles, users_feature, bundles_feature):
    pred_flat, parts = _sc_kernel(
        users.reshape(_B), bundles.reshape(_B * _L),
        users_feature, bundles_feature)
    pred = pred_flat.reshape(_B, _L)
    loss = jnp.float32(_L2) * jnp.sum(parts)
    return (pred, loss)
